# baseline (speedup 1.0000x reference)
import functools

import numpy as np
import jax
import jax.numpy as jnp
from jax import lax
from jax.experimental import pallas as pl
from jax.experimental.pallas import tpu as pltpu
from jax.experimental.pallas import tpu_sc as plsc

_D = 64
_NC, _NS = 2, 16
_NW = _NC * _NS
_EPS = 1e-12
_L = 16


@functools.lru_cache(maxsize=None)
def _pe_const(seq_len: int):
    position = np.arange(0, seq_len, dtype=np.float32)[:, None]
    div_term = np.exp(np.arange(0, _D, 2, dtype=np.float32) * -(np.log(10000.0) / _D))
    pe = np.zeros((seq_len, _D), dtype=np.float32)
    pe[:, 0::2] = np.sin(position * div_term)
    pe[:, 1::2] = np.cos(position * div_term)
    return pe


@functools.lru_cache(maxsize=None)
def _fused_call(n_rows: int, seq: int, chunk: int):
    assert n_rows % (_NW * chunk) == 0 and chunk % _L == 0
    n_chunks = n_rows // (_NW * chunk)
    rows_per_w = n_chunks * chunk
    n_groups = chunk // _L
    mesh = plsc.VectorSubcoreMesh(
        core_axis_name="c", subcore_axis_name="s",
        num_cores=_NC, num_subcores=_NS)

    @functools.partial(
        pl.kernel,
        out_type=jax.ShapeDtypeStruct((n_rows, _D), jnp.float32),
        mesh=mesh,
        scratch_types=[
            pltpu.VMEM((chunk,), jnp.int32),
            pltpu.VMEM((chunk, _D), jnp.float32),
            pltpu.VMEM((chunk, _D), jnp.float32),
            pltpu.VMEM((seq, _D), jnp.float32),
            pltpu.VMEM((_D,), jnp.float32),
            pltpu.VMEM((_D,), jnp.float32),
            pltpu.VMEM((_D * _L,), jnp.float32),
            pltpu.VMEM((_D * _L,), jnp.float32),
            pltpu.VMEM((_D * _L,), jnp.float32),
            pltpu.SemaphoreType.DMA,
        ],
        compiler_params=pltpu.CompilerParams(
            use_tc_tiling_on_sc=False, needs_layout_passes=False),
    )
    def k(idx_hbm, table_hbm, pe_hbm, gamma_hbm, beta_hbm, out_hbm,
          idx_v, rows_v, out_v, pe_v, gamma_v, beta_v, e_scr, g_scr, b_scr,
          sem):
        wid = lax.axis_index("s") * _NC + lax.axis_index("c")
        base0 = wid * rows_per_w
        iota = lax.iota(jnp.int32, _L)

        pltpu.sync_copy(pe_hbm, pe_v)
        pltpu.sync_copy(gamma_hbm, gamma_v)
        pltpu.sync_copy(beta_hbm, beta_v)
        for d in range(_D):
            dd = jnp.full((_L,), d, jnp.int32)
            g_scr[pl.ds(d * _L, _L)] = plsc.load_gather(gamma_v, [dd])
            b_scr[pl.ds(d * _L, _L)] = plsc.load_gather(beta_v, [dd])

        def chunk_body(i, carry):
            hbase = base0 + i * chunk
            pltpu.sync_copy(idx_hbm.at[pl.ds(hbase, chunk)], idx_v)
            pltpu.async_copy(table_hbm.at[idx_v], rows_v, sem).wait()

            def group_body(g, carry2):
                tok = g * _L + iota
                s_idx = lax.rem(hbase + tok, seq)
                acc = jnp.zeros((_L,), jnp.float32)
                acc2 = jnp.zeros((_L,), jnp.float32)
                for d in range(_D):
                    dd = jnp.full((_L,), d, jnp.int32)
                    x = plsc.load_gather(rows_v, [tok, dd])
                    p = plsc.load_gather(pe_v, [s_idx, dd])
                    e = x + p
                    e_scr[pl.ds(d * _L, _L)] = e
                    acc = acc + e
                    acc2 = acc2 + e * e
                mu = acc * (1.0 / _D)
                var = acc2 * (1.0 / _D) - mu * mu + _EPS
                y = plsc.bitcast(
                    jnp.int32(0x5F3759DF) - (plsc.bitcast(var, jnp.int32) >> 1),
                    jnp.float32)
                for _ in range(3):
                    y = y * (1.5 - 0.5 * var * y * y)
                for d in range(_D):
                    dd = jnp.full((_L,), d, jnp.int32)
                    e = e_scr[pl.ds(d * _L, _L)]
                    gd = g_scr[pl.ds(d * _L, _L)]
                    bd = b_scr[pl.ds(d * _L, _L)]
                    plsc.store_scatter(out_v, [tok, dd],
                                       (e - mu) * y * gd + bd)
                return carry2

            lax.fori_loop(0, n_groups, group_body, 0, unroll=False)
            pltpu.sync_copy(out_v, out_hbm.at[pl.ds(hbase, chunk)])
            return carry

        lax.fori_loop(0, n_chunks, chunk_body, 0, unroll=False)

    return k


def kernel(input_ids, table, gamma, beta):
    B, S = input_ids.shape
    ids_flat = input_ids.reshape(-1).astype(jnp.int32)
    pe = jnp.asarray(_pe_const(S))
    out = _fused_call(B * S, S, 640)(ids_flat, table, pe, gamma, beta)
    return out.reshape(B, S, _D)


# speedup vs baseline: 1.8965x; 1.8965x over previous
import functools

import numpy as np
import jax
import jax.numpy as jnp
from jax import lax
from jax.experimental import pallas as pl
from jax.experimental.pallas import tpu as pltpu
from jax.experimental.pallas import tpu_sc as plsc

_D = 64
_NC, _NS = 2, 16
_NW = _NC * _NS
_EPS = 1e-12


@functools.lru_cache(maxsize=None)
def _pe_const(seq_len: int):
    position = np.arange(0, seq_len, dtype=np.float32)[:, None]
    div_term = np.exp(np.arange(0, _D, 2, dtype=np.float32) * -(np.log(10000.0) / _D))
    pe = np.zeros((seq_len, _D), dtype=np.float32)
    pe[:, 0::2] = np.sin(position * div_term)
    pe[:, 1::2] = np.cos(position * div_term)
    return pe


@functools.lru_cache(maxsize=None)
def _gather_call(n_rows: int, chunk: int):
    assert n_rows % (_NW * chunk) == 0 and chunk % 8 == 0
    n_chunks = n_rows // (_NW * chunk)
    rows_per_w = n_chunks * chunk
    mesh = plsc.VectorSubcoreMesh(
        core_axis_name="c", subcore_axis_name="s",
        num_cores=_NC, num_subcores=_NS)

    @functools.partial(
        pl.kernel,
        out_type=jax.ShapeDtypeStruct((n_rows, _D), jnp.float32),
        mesh=mesh,
        scratch_types=[
            pltpu.VMEM((chunk,), jnp.int32),
            pltpu.VMEM((chunk, _D), jnp.float32),
            pltpu.SemaphoreType.DMA,
        ],
        compiler_params=pltpu.CompilerParams(use_tc_tiling_on_sc=False),
    )
    def k(idx_hbm, table_hbm, out_hbm, idx_v, rows_v, sem):
        wid = lax.axis_index("s") * _NC + lax.axis_index("c")
        base0 = wid * rows_per_w

        def body(i, carry):
            base = base0 + i * chunk
            pltpu.sync_copy(idx_hbm.at[pl.ds(base, chunk)], idx_v)
            pltpu.async_copy(table_hbm.at[idx_v], rows_v, sem).wait()
            pltpu.sync_copy(rows_v, out_hbm.at[pl.ds(base, chunk)])
            return carry

        lax.fori_loop(0, n_chunks, body, 0, unroll=False)

    return k


def _ln2_body(x_ref, pe_ref, g_ref, b_ref, o_ref):
    x = x_ref[...]
    bb = x.shape[0]
    reps = bb // pe_ref.shape[0]
    pe = pe_ref[...]
    pe = jnp.broadcast_to(pe[None], (reps,) + pe.shape).reshape(bb, 128)
    x = x + pe
    xl, xh = x[:, :_D], x[:, _D:]
    ml = jnp.mean(xl, axis=-1, keepdims=True)
    mh = jnp.mean(xh, axis=-1, keepdims=True)
    mu = jnp.concatenate(
        [jnp.broadcast_to(ml, (bb, _D)), jnp.broadcast_to(mh, (bb, _D))], -1)
    xc = x - mu
    sq = xc * xc
    vl = jnp.mean(sq[:, :_D], axis=-1, keepdims=True)
    vh = jnp.mean(sq[:, _D:], axis=-1, keepdims=True)
    var = jnp.concatenate(
        [jnp.broadcast_to(vl, (bb, _D)), jnp.broadcast_to(vh, (bb, _D))], -1)
    o_ref[...] = xc * lax.rsqrt(var + _EPS) * g_ref[...] + b_ref[...]


@functools.lru_cache(maxsize=None)
def _ln2_call(n_pairs: int, pe_rows: int, block: int):
    grid = (n_pairs // block,)
    return pl.pallas_call(
        _ln2_body,
        grid=grid,
        in_specs=[
            pl.BlockSpec((block, 2 * _D), lambda i: (i, 0)),
            pl.BlockSpec((pe_rows, 2 * _D), lambda i: (0, 0)),
            pl.BlockSpec((1, 2 * _D), lambda i: (0, 0)),
            pl.BlockSpec((1, 2 * _D), lambda i: (0, 0)),
        ],
        out_specs=pl.BlockSpec((block, 2 * _D), lambda i: (i, 0)),
        out_shape=jax.ShapeDtypeStruct((n_pairs, 2 * _D), jnp.float32),
    )


def kernel(input_ids, table, gamma, beta):
    B, S = input_ids.shape
    n = B * S
    ids_flat = input_ids.reshape(-1).astype(jnp.int32)
    emb = _gather_call(n, 800)(ids_flat, table)
    emb2 = emb.reshape(n // 2, 2 * _D)
    pe2 = jnp.asarray(_pe_const(S)).reshape(S // 2, 2 * _D)
    g2 = jnp.concatenate([gamma, gamma]).reshape(1, 2 * _D)
    b2 = jnp.concatenate([beta, beta]).reshape(1, 2 * _D)
    out2 = _ln2_call(n // 2, S // 2, 800)(emb2, pe2, g2, b2)
    return out2.reshape(B, S, _D)


# speedup vs baseline: 3.0131x; 1.5888x over previous
import functools

import numpy as np
import jax
import jax.numpy as jnp
from jax import lax
from jax.experimental import pallas as pl
from jax.experimental.pallas import tpu as pltpu
from jax.experimental.pallas import tpu_sc as plsc

_D = 64
_NC, _NS = 2, 16
_NW = _NC * _NS
_EPS = 1e-12
_TW = 4096


@functools.lru_cache(maxsize=None)
def _pe_const(seq_len: int):
    position = np.arange(0, seq_len, dtype=np.float32)[:, None]
    div_term = np.exp(np.arange(0, _D, 2, dtype=np.float32) * -(np.log(10000.0) / _D))
    pe = np.zeros((seq_len, _D), dtype=np.float32)
    pe[:, 0::2] = np.sin(position * div_term)
    pe[:, 1::2] = np.cos(position * div_term)
    return pe


def _tr_body(x1_ref, x2_ref, o_ref):
    o_ref[...] = jnp.concatenate([x1_ref[...].T, x2_ref[...].T], axis=1)


@functools.lru_cache(maxsize=None)
def _tr_call(vocab: int, half: int):
    grid = half // _TW
    last = (vocab - 1) // _TW
    return pl.pallas_call(
        _tr_body,
        grid=(grid,),
        in_specs=[
            pl.BlockSpec((_D, _TW), lambda i: (0, i)),
            pl.BlockSpec((_D, _TW), lambda i: (0, jnp.minimum(i + grid, last))),
        ],
        out_specs=pl.BlockSpec((_TW, 2 * _D), lambda i: (i, 0)),
        out_shape=jax.ShapeDtypeStruct((half, 2 * _D), jnp.float32),
    )


@functools.lru_cache(maxsize=None)
def _gather_call(n_rows: int, table_rows: int, chunk: int):
    assert n_rows % (_NW * chunk) == 0 and chunk % 8 == 0
    n_chunks = n_rows // (_NW * chunk)
    rows_per_w = n_chunks * chunk
    mesh = plsc.VectorSubcoreMesh(
        core_axis_name="c", subcore_axis_name="s",
        num_cores=_NC, num_subcores=_NS)

    @functools.partial(
        pl.kernel,
        out_type=jax.ShapeDtypeStruct((n_rows, _D), jnp.float32),
        mesh=mesh,
        scratch_types=[
            pltpu.VMEM((chunk,), jnp.int32),
            pltpu.VMEM((chunk, _D), jnp.float32),
            pltpu.SemaphoreType.DMA,
        ],
        compiler_params=pltpu.CompilerParams(use_tc_tiling_on_sc=False),
    )
    def k(idx_hbm, table_hbm, out_hbm, idx_v, rows_v, sem):
        wid = lax.axis_index("s") * _NC + lax.axis_index("c")
        base0 = wid * rows_per_w

        def body(i, carry):
            base = base0 + i * chunk
            pltpu.sync_copy(idx_hbm.at[pl.ds(base, chunk)], idx_v)
            pltpu.async_copy(table_hbm.at[idx_v], rows_v, sem).wait()
            pltpu.sync_copy(rows_v, out_hbm.at[pl.ds(base, chunk)])
            return carry

        lax.fori_loop(0, n_chunks, body, 0, unroll=False)

    return k


def _ln2_body(x_ref, pe_ref, g_ref, b_ref, o_ref):
    x = x_ref[...] + pe_ref[...]
    bb = x.shape[0]
    xl, xh = x[:, :_D], x[:, _D:]
    ml = jnp.mean(xl, axis=-1, keepdims=True)
    mh = jnp.mean(xh, axis=-1, keepdims=True)
    mu = jnp.concatenate(
        [jnp.broadcast_to(ml, (bb, _D)), jnp.broadcast_to(mh, (bb, _D))], -1)
    xc = x - mu
    sq = xc * xc
    vl = jnp.mean(sq[:, :_D], axis=-1, keepdims=True)
    vh = jnp.mean(sq[:, _D:], axis=-1, keepdims=True)
    var = jnp.concatenate(
        [jnp.broadcast_to(vl, (bb, _D)), jnp.broadcast_to(vh, (bb, _D))], -1)
    o_ref[...] = xc * lax.rsqrt(var + _EPS) * g_ref[...] + b_ref[...]


@functools.lru_cache(maxsize=None)
def _ln2_call(n_pairs: int, block: int):
    grid = (n_pairs // block,)
    return pl.pallas_call(
        _ln2_body,
        grid=grid,
        in_specs=[
            pl.BlockSpec((block, 2 * _D), lambda i: (i, 0)),
            pl.BlockSpec((block, 2 * _D), lambda i: (0, 0)),
            pl.BlockSpec((1, 2 * _D), lambda i: (0, 0)),
            pl.BlockSpec((1, 2 * _D), lambda i: (0, 0)),
        ],
        out_specs=pl.BlockSpec((block, 2 * _D), lambda i: (i, 0)),
        out_shape=jax.ShapeDtypeStruct((n_pairs, 2 * _D), jnp.float32),
    )


def kernel(input_ids, table, gamma, beta):
    B, S = input_ids.shape
    n = B * S
    V = table.shape[0]
    half = ((V // 2) // _TW + 1) * _TW

    t2 = _tr_call(V, half)(table.T, table.T)
    t_lin = t2.reshape(2 * half, _D)

    ids = input_ids.reshape(-1).astype(jnp.int32)
    idx = jnp.where(ids < half, 2 * ids, 2 * (ids - half) + 1)

    emb = _gather_call(n, 2 * half, 800)(idx, t_lin)
    emb2 = emb.reshape(n // 2, 2 * _D)

    block = 800
    pe2 = jnp.asarray(_pe_const(S)).reshape(S // 2, 2 * _D)
    pe_blk = jnp.tile(pe2, (block // (S // 2), 1))
    g2 = jnp.concatenate([gamma, gamma]).reshape(1, 2 * _D)
    b2 = jnp.concatenate([beta, beta]).reshape(1, 2 * _D)
    out2 = _ln2_call(n // 2, block)(emb2, pe_blk, g2, b2)
    return out2.reshape(B, S, _D)
